# baseline (speedup 1.0000x reference)
import jax
import jax.numpy as jnp
from jax.experimental import pallas as pl

_TOKEN_BLOCK = 512
_KSPLIT = 8


def _router_body(x_ref, w_ref, b_ref, o_ref):
    d = x_ref.shape[1]
    kc = d // _KSPLIT
    dn = (((1,), (1,)), ((), ()))
    acc = b_ref[...]
    for k in range(_KSPLIT):
        acc = acc + jax.lax.dot_general(
            x_ref[:, pl.ds(k * kc, kc)], w_ref[:, pl.ds(k * kc, kc)],
            dimension_numbers=dn, preferred_element_type=jnp.float32)
    o_ref[...] = acc


def kernel(x, W, b):
    tokens, d = x.shape
    n_experts = W.shape[0]
    blk = _TOKEN_BLOCK
    return pl.pallas_call(
        _router_body,
        grid=(tokens // blk,),
        in_specs=[
            pl.BlockSpec((blk, d), lambda i: (i, 0)),
            pl.BlockSpec((n_experts, d), lambda i: (0, 0)),
            pl.BlockSpec((1, n_experts), lambda i: (0, 0)),
        ],
        out_specs=pl.BlockSpec((blk, n_experts), lambda i: (i, 0)),
        out_shape=jax.ShapeDtypeStruct((tokens, n_experts), jnp.float32),
    )(x, W, b.reshape(1, n_experts))
